# Initial kernel scaffold; baseline (speedup 1.0000x reference)
import jax
import jax.numpy as jnp
from jax.experimental import pallas as pl


def kernel(positions, node_attrs, shifts, eps, w_embed, w_r1_0, w_r2_0, w_r1_1, w_r2_1, w_sc_0, w_sc_1, w_read_0, w_read_1, edge_index, batch):
    raise NotImplementedError("write your pallas kernel here")


# speedup vs baseline: 3.9336x; 3.9336x over previous
import functools

import jax
import jax.numpy as jnp
from jax.experimental import pallas as pl

N = 10000
E = 160000
NE = 4
F = 32
L2 = 9
NB = 8
T = 1000
R_MAX = 5.0
AVG_NEI = 16.0
NOUT = NE + 3
T_IDX = 500

EB = 1600


def _edge_messages_body(pos_s_ref, pos_r_ref, sh_ref, h_ref, w_r1_ref, w_r2_ref, m_ref):
    vec = pos_r_ref[...] - pos_s_ref[...] + sh_ref[...]
    r2 = jnp.sum(vec * vec, axis=1)
    r = jnp.sqrt(r2) + 1e-9
    inv_r = 1.0 / r
    u = vec * inv_r[:, None]
    x, y, z = u[:, 0], u[:, 1], u[:, 2]
    c1 = jnp.sqrt(3.0)
    c2 = jnp.sqrt(15.0)
    one = jnp.ones_like(x)
    sh = jnp.stack([
        one,
        c1 * x, c1 * y, c1 * z,
        c2 * x * y,
        c2 * y * z,
        (jnp.sqrt(5.0) / 2.0) * (3.0 * z * z - 1.0),
        c2 * x * z,
        (c2 / 2.0) * (x * x - y * y),
    ], axis=-1)

    n = jax.lax.broadcasted_iota(jnp.int32, (NB,), 0).astype(jnp.float32) + 1.0
    bessel = jnp.sqrt(2.0 / R_MAX) * jnp.sin(n[None, :] * (jnp.pi / R_MAX) * r[:, None]) * inv_r[:, None]
    p = 5.0
    xr = r / R_MAX
    xp = xr ** 5
    env = (1.0 - ((p + 1.0) * (p + 2.0) / 2.0) * xp
           + p * (p + 2.0) * xp * xr
           - (p * (p + 1.0) / 2.0) * xp * xr * xr)
    env = jnp.where(xr < 1.0, env, 0.0)
    ef = bessel * env[:, None]

    pre = jnp.dot(ef, w_r1_ref[...], preferred_element_type=jnp.float32)
    zact = pre * jax.nn.sigmoid(pre)
    tpw = jnp.dot(zact, w_r2_ref[...], preferred_element_type=jnp.float32)
    h = h_ref[...]
    m = (h[:, :, None] * sh[:, None, :]) * tpw.reshape(-1, F, L2)
    m_ref[...] = m.reshape(-1, F * L2)


@functools.partial(jax.jit, static_argnames=())
def _edge_messages(pos_s, pos_r, shifts, h_gather, w_r1, w_r2):
    grid = (E // EB,)
    return pl.pallas_call(
        _edge_messages_body,
        grid=grid,
        in_specs=[
            pl.BlockSpec((EB, 3), lambda i: (i, 0)),
            pl.BlockSpec((EB, 3), lambda i: (i, 0)),
            pl.BlockSpec((EB, 3), lambda i: (i, 0)),
            pl.BlockSpec((EB, F), lambda i: (i, 0)),
            pl.BlockSpec((NB, F), lambda i: (0, 0)),
            pl.BlockSpec((F, F * L2), lambda i: (0, 0)),
        ],
        out_specs=pl.BlockSpec((EB, F * L2), lambda i: (i, 0)),
        out_shape=jax.ShapeDtypeStruct((E, F * L2), jnp.float32),
    )(pos_s, pos_r, shifts, h_gather, w_r1, w_r2)


def kernel(positions, node_attrs, shifts, eps, w_embed, w_r1_0, w_r2_0, w_r1_1, w_r2_1,
           w_sc_0, w_sc_1, w_read_0, w_read_1, edge_index, batch):
    alphas = 1.0 - jnp.linspace(1e-4, 0.02, T)
    abar = jnp.cumprod(alphas)[T_IDX]
    s = jnp.sqrt(abar)
    sq = jnp.sqrt(1.0 - abar)
    node_attrs = node_attrs / 4.0
    pos_n = s * positions + sq * eps[:, -3:]
    attrs_n = s * node_attrs + sq * eps[:, :NE]
    t_feat = jnp.full((node_attrs.shape[0], 1), T_IDX / float(T), dtype=jnp.float32)
    h0 = jnp.concatenate([attrs_n, t_feat], axis=-1) @ w_embed
    node_feats = jnp.zeros((N, F, L2), dtype=jnp.float32).at[:, :, 0].set(h0)

    sender = edge_index[0]
    receiver = edge_index[1]
    pos_s = pos_n[sender]
    pos_r = pos_n[receiver]

    preds_sum = jnp.zeros((N, NOUT), dtype=jnp.float32)
    for w_r1, w_r2, w_sc, w_read in ((w_r1_0, w_r2_0, w_sc_0, w_read_0),
                                     (w_r1_1, w_r2_1, w_sc_1, w_read_1)):
        h_gather = node_feats[:, :, 0][sender]
        m = _edge_messages(pos_s, pos_r, shifts, h_gather, w_r1, w_r2)
        agg = jnp.zeros((N, F * L2), dtype=jnp.float32).at[receiver].add(m) / AVG_NEI
        agg = agg.reshape(N, F, L2)
        sc = jnp.einsum("nfl,fg->ngl", node_feats, w_sc)
        nf = agg + sc
        inv = jnp.sum(nf ** 2, axis=-1)
        nf = nf * (1.0 + 0.1 * jnp.tanh(inv))[:, :, None]
        node_feats = nf
        preds_sum = preds_sum + nf.reshape(N, F * L2) @ w_read

    pn_pos = preds_sum[:, -3:]
    pn_lab = preds_sum[:, :-3]
    err_pos = (pn_pos - eps[:, -3:]) ** 2
    err_lab = (pn_lab - eps[:, :NE]) ** 2
    lp = jax.ops.segment_sum(err_pos, batch, num_segments=1).sum(axis=-1)
    ll = jax.ops.segment_sum(err_lab, batch, num_segments=1).sum(axis=-1)
    num_nodes = jnp.array([float(N)], dtype=jnp.float32)
    loss = 0.5 * (lp + ll) / (num_nodes * (3.0 + NE))
    return (pn_lab, pn_pos, eps[:, :NE], eps[:, -3:], loss)


# speedup vs baseline: 14.1520x; 3.5977x over previous
import functools

import jax
import jax.numpy as jnp
from jax import lax
from jax.experimental import pallas as pl
from jax.experimental.pallas import tpu as pltpu
from jax.experimental.pallas import tpu_sc as plsc

N = 10000
E = 160000
NE = 4
F = 32
L2 = 9
NB = 8
T = 1000
R_MAX = 5.0
AVG_NEI = 16.0
NOUT = NE + 3
T_IDX = 500

NC = 2
NS = 16
NW = NC * NS
CH = 128
E_PAD = 163840
EPW = E_PAD // NW
NCHUNK = EPW // CH

EB = 1024

_MESH = plsc.VectorSubcoreMesh(core_axis_name="c", subcore_axis_name="s")
_SC_PARAMS = pltpu.CompilerParams(use_tc_tiling_on_sc=False)


def _gather_geom_body(pos_tbl, snd, rcv, out_s, out_r, idx_s, idx_r, rows_s, rows_r, sem):
    wid = lax.axis_index("s") * NC + lax.axis_index("c")
    base = wid * EPW
    pltpu.sync_copy(snd.at[pl.ds(base, EPW)], idx_s)
    pltpu.sync_copy(rcv.at[pl.ds(base, EPW)], idx_r)
    for j in range(NCHUNK):
        pltpu.async_copy(pos_tbl.at[idx_s.at[pl.ds(j * CH, CH)]], rows_s, sem).wait()
        pltpu.sync_copy(rows_s, out_s.at[pl.ds(base + j * CH, CH)])
        pltpu.async_copy(pos_tbl.at[idx_r.at[pl.ds(j * CH, CH)]], rows_r, sem).wait()
        pltpu.sync_copy(rows_r, out_r.at[pl.ds(base + j * CH, CH)])


_gather_geom = functools.partial(
    pl.kernel,
    mesh=_MESH,
    out_type=[
        jax.ShapeDtypeStruct((E_PAD, 16), jnp.float32),
        jax.ShapeDtypeStruct((E_PAD, 16), jnp.float32),
    ],
    scratch_types=[
        pltpu.VMEM((EPW,), jnp.int32),
        pltpu.VMEM((EPW,), jnp.int32),
        pltpu.VMEM((CH, 16), jnp.float32),
        pltpu.VMEM((CH, 16), jnp.float32),
        pltpu.SemaphoreType.DMA,
    ],
    compiler_params=_SC_PARAMS,
)(_gather_geom_body)


def _gather_h_body(h_tbl, snd, out_h, idx_s, rows_h, sem):
    wid = lax.axis_index("s") * NC + lax.axis_index("c")
    base = wid * EPW
    pltpu.sync_copy(snd.at[pl.ds(base, EPW)], idx_s)
    for j in range(NCHUNK):
        pltpu.async_copy(h_tbl.at[idx_s.at[pl.ds(j * CH, CH)]], rows_h, sem).wait()
        pltpu.sync_copy(rows_h, out_h.at[pl.ds(base + j * CH, CH)])


_gather_h = functools.partial(
    pl.kernel,
    mesh=_MESH,
    out_type=jax.ShapeDtypeStruct((E_PAD, F), jnp.float32),
    scratch_types=[
        pltpu.VMEM((EPW,), jnp.int32),
        pltpu.VMEM((CH, F), jnp.float32),
        pltpu.SemaphoreType.DMA,
    ],
    compiler_params=_SC_PARAMS,
)(_gather_h_body)


def _edge_messages_body(pos_s_ref, pos_r_ref, h_ref, w_r1_ref, w_r2_ref, m_ref):
    psT = jnp.transpose(pos_s_ref[...])
    prT = jnp.transpose(pos_r_ref[...])
    dx = prT[0] - psT[0]
    dy = prT[1] - psT[1]
    dz = prT[2] - psT[2]
    r2 = dx * dx + dy * dy + dz * dz
    r = jnp.sqrt(r2) + 1e-9
    inv_r = 1.0 / r
    x, y, z = dx * inv_r, dy * inv_r, dz * inv_r
    c1 = jnp.sqrt(3.0)
    c2 = jnp.sqrt(15.0)
    one = jnp.ones_like(x)
    shT = jnp.stack([
        one,
        c1 * x, c1 * y, c1 * z,
        c2 * x * y,
        c2 * y * z,
        (jnp.sqrt(5.0) / 2.0) * (3.0 * z * z - 1.0),
        c2 * x * z,
        (c2 / 2.0) * (x * x - y * y),
    ], axis=0)

    theta = (jnp.pi / R_MAX) * r
    s1 = jnp.sin(theta)
    c1t = jnp.cos(theta)
    two_c = 2.0 * c1t
    sins = [s1, two_c * s1]
    for _ in range(2, NB):
        sins.append(two_c * sins[-1] - sins[-2])
    p = 5.0
    xr = r / R_MAX
    xp = xr ** 5
    env = (1.0 - ((p + 1.0) * (p + 2.0) / 2.0) * xp
           + p * (p + 2.0) * xp * xr
           - (p * (p + 1.0) / 2.0) * xp * xr * xr)
    env = jnp.where(xr < 1.0, env, 0.0)
    scale = jnp.sqrt(2.0 / R_MAX) * inv_r * env
    efT = jnp.stack([s * scale for s in sins], axis=0)

    sh = jnp.transpose(shT)
    ef = jnp.transpose(efT)
    pre = jnp.dot(ef, w_r1_ref[...], preferred_element_type=jnp.float32)
    zact = pre * jax.nn.sigmoid(pre)
    tpw = jnp.dot(zact, w_r2_ref[...], preferred_element_type=jnp.float32)
    h = h_ref[...]
    col = lax.broadcasted_iota(jnp.int32, (F, F * L2), 1)
    row = lax.broadcasted_iota(jnp.int32, (F, F * L2), 0)
    rep = (col // L2 == row).astype(jnp.float32)
    col9 = lax.broadcasted_iota(jnp.int32, (L2, F * L2), 1)
    row9 = lax.broadcasted_iota(jnp.int32, (L2, F * L2), 0)
    til = (col9 % L2 == row9).astype(jnp.float32)
    h_rep = jnp.dot(h, rep, preferred_element_type=jnp.float32)
    sh_til = jnp.dot(sh, til, preferred_element_type=jnp.float32)
    m_ref[...] = h_rep * sh_til * tpw


def _edge_messages(pos_s, pos_r, h_gather, w_r1, w_r2):
    grid = (E_PAD // EB,)
    return pl.pallas_call(
        _edge_messages_body,
        grid=grid,
        in_specs=[
            pl.BlockSpec((EB, 16), lambda i: (i, 0)),
            pl.BlockSpec((EB, 16), lambda i: (i, 0)),
            pl.BlockSpec((EB, F), lambda i: (i, 0)),
            pl.BlockSpec((NB, F), lambda i: (0, 0)),
            pl.BlockSpec((F, F * L2), lambda i: (0, 0)),
        ],
        out_specs=pl.BlockSpec((EB, F * L2), lambda i: (i, 0)),
        out_shape=jax.ShapeDtypeStruct((E_PAD, F * L2), jnp.float32),
    )(pos_s, pos_r, h_gather, w_r1, w_r2)


def kernel(positions, node_attrs, shifts, eps, w_embed, w_r1_0, w_r2_0, w_r1_1, w_r2_1,
           w_sc_0, w_sc_1, w_read_0, w_read_1, edge_index, batch):
    alphas = 1.0 - jnp.linspace(1e-4, 0.02, T)
    abar = jnp.cumprod(alphas)[T_IDX]
    s = jnp.sqrt(abar)
    sq = jnp.sqrt(1.0 - abar)
    node_attrs = node_attrs / 4.0
    pos_n = s * positions + sq * eps[:, -3:]
    attrs_n = s * node_attrs + sq * eps[:, :NE]
    t_feat = jnp.full((node_attrs.shape[0], 1), T_IDX / float(T), dtype=jnp.float32)
    h0 = jnp.concatenate([attrs_n, t_feat], axis=-1) @ w_embed
    node_feats = jnp.zeros((N, F, L2), dtype=jnp.float32).at[:, :, 0].set(h0)

    sender = edge_index[0]
    receiver = edge_index[1]
    pad = E_PAD - E
    snd_pad = jnp.concatenate([sender, jnp.zeros((pad,), jnp.int32)])
    rcv_pad = jnp.concatenate([receiver, jnp.zeros((pad,), jnp.int32)])
    rcv_scatter = jnp.concatenate([receiver, jnp.full((pad,), N, jnp.int32)])

    pos_tbl = jnp.zeros((N, 16), jnp.float32).at[:, :3].set(pos_n)
    pos_s16, pos_r16 = _gather_geom(pos_tbl, snd_pad, rcv_pad)

    preds_sum = jnp.zeros((N, NOUT), dtype=jnp.float32)
    for w_r1, w_r2, w_sc, w_read in ((w_r1_0, w_r2_0, w_sc_0, w_read_0),
                                     (w_r1_1, w_r2_1, w_sc_1, w_read_1)):
        h_tbl = node_feats[:, :, 0]
        h_gather = _gather_h(h_tbl, snd_pad)
        m = _edge_messages(pos_s16, pos_r16, h_gather, w_r1, w_r2)
        agg = jnp.zeros((N + 1, F * L2), dtype=jnp.float32).at[rcv_scatter].add(m)[:N] / AVG_NEI
        agg = agg.reshape(N, F, L2)
        sc = jnp.einsum("nfl,fg->ngl", node_feats, w_sc)
        nf = agg + sc
        inv = jnp.sum(nf ** 2, axis=-1)
        nf = nf * (1.0 + 0.1 * jnp.tanh(inv))[:, :, None]
        node_feats = nf
        preds_sum = preds_sum + nf.reshape(N, F * L2) @ w_read

    pn_pos = preds_sum[:, -3:]
    pn_lab = preds_sum[:, :-3]
    err_pos = (pn_pos - eps[:, -3:]) ** 2
    err_lab = (pn_lab - eps[:, :NE]) ** 2
    lp = jax.ops.segment_sum(err_pos, batch, num_segments=1).sum(axis=-1)
    ll = jax.ops.segment_sum(err_lab, batch, num_segments=1).sum(axis=-1)
    num_nodes = jnp.array([float(N)], dtype=jnp.float32)
    loss = 0.5 * (lp + ll) / (num_nodes * (3.0 + NE))
    return (pn_lab, pn_pos, eps[:, :NE], eps[:, -3:], loss)


# speedup vs baseline: 14.2984x; 1.0103x over previous
import functools

import jax
import jax.numpy as jnp
from jax import lax
from jax.experimental import pallas as pl
from jax.experimental.pallas import tpu as pltpu
from jax.experimental.pallas import tpu_sc as plsc

N = 10000
E = 160000
NE = 4
F = 32
L2 = 9
NB = 8
T = 1000
R_MAX = 5.0
AVG_NEI = 16.0
NOUT = NE + 3
T_IDX = 500

NC = 2
NS = 16
NW = NC * NS
CH = 128
E_PAD = 163840
EPW = E_PAD // NW
NCHUNK = EPW // CH

EB = 1024

_MESH = plsc.VectorSubcoreMesh(core_axis_name="c", subcore_axis_name="s")
_SC_PARAMS = pltpu.CompilerParams(use_tc_tiling_on_sc=False)


def _gather_geom_body(pos_tbl, snd, rcv, out_s, out_r, idx_s, idx_r, rows_s, rows_r, sem):
    wid = lax.axis_index("s") * NC + lax.axis_index("c")
    base = wid * EPW
    pltpu.sync_copy(snd.at[pl.ds(base, EPW)], idx_s)
    pltpu.sync_copy(rcv.at[pl.ds(base, EPW)], idx_r)
    for j in range(NCHUNK):
        pltpu.async_copy(pos_tbl.at[idx_s.at[pl.ds(j * CH, CH)]], rows_s, sem).wait()
        pltpu.sync_copy(rows_s, out_s.at[pl.ds(base + j * CH, CH)])
        pltpu.async_copy(pos_tbl.at[idx_r.at[pl.ds(j * CH, CH)]], rows_r, sem).wait()
        pltpu.sync_copy(rows_r, out_r.at[pl.ds(base + j * CH, CH)])


_gather_geom = functools.partial(
    pl.kernel,
    mesh=_MESH,
    out_type=[
        jax.ShapeDtypeStruct((E_PAD, 16), jnp.float32),
        jax.ShapeDtypeStruct((E_PAD, 16), jnp.float32),
    ],
    scratch_types=[
        pltpu.VMEM((EPW,), jnp.int32),
        pltpu.VMEM((EPW,), jnp.int32),
        pltpu.VMEM((CH, 16), jnp.float32),
        pltpu.VMEM((CH, 16), jnp.float32),
        pltpu.SemaphoreType.DMA,
    ],
    compiler_params=_SC_PARAMS,
)(_gather_geom_body)


def _gather_h_body(h_tbl, snd, out_h, idx_s, rows_h, sem):
    wid = lax.axis_index("s") * NC + lax.axis_index("c")
    base = wid * EPW
    pltpu.sync_copy(snd.at[pl.ds(base, EPW)], idx_s)
    for j in range(NCHUNK):
        pltpu.async_copy(h_tbl.at[idx_s.at[pl.ds(j * CH, CH)]], rows_h, sem).wait()
        pltpu.sync_copy(rows_h, out_h.at[pl.ds(base + j * CH, CH)])


_gather_h = functools.partial(
    pl.kernel,
    mesh=_MESH,
    out_type=jax.ShapeDtypeStruct((E_PAD, F), jnp.float32),
    scratch_types=[
        pltpu.VMEM((EPW,), jnp.int32),
        pltpu.VMEM((CH, F), jnp.float32),
        pltpu.SemaphoreType.DMA,
    ],
    compiler_params=_SC_PARAMS,
)(_gather_h_body)


HALF = 5120
TRASH = 256
ACC_ROWS = HALF + TRASH
ES = E_PAD // NS
SCH = ES // CH
ZROWS = ACC_ROWS // NS
OROWS = HALF // NS


SCCH = 80
SSCH = ES // SCCH


def _scatter_body(m_hbm, rcv_hbm, zeros_hbm, out_hbm, ridx_v, loc_v, m_v, acc, sem):
    c = lax.axis_index("c")
    s = lax.axis_index("s")
    pltpu.sync_copy(zeros_hbm.at[pl.ds(s * ZROWS, ZROWS)], acc.at[pl.ds(s * ZROWS, ZROWS)])
    base = s * ES
    half_base = c * HALF
    iota16 = lax.broadcasted_iota(jnp.int32, (16,), 0)
    plsc.subcore_barrier()
    for j in range(SSCH):
        pltpu.sync_copy(rcv_hbm.at[pl.ds(base + j * SCCH, SCCH)], ridx_v)
        pltpu.sync_copy(m_hbm.at[pl.ds(base + j * SCCH, SCCH)], m_v)

        def fill(g, carry, j=j):
            iv = ridx_v[pl.ds(g * 16, 16)]
            loc = iv - half_base
            own = (loc >= 0) & (loc < HALF)
            trash = HALF + (((j * SCCH + g * 16) + iota16) & (TRASH - 1))
            loc_v[pl.ds(g * 16, 16)] = jnp.where(own, loc, trash)
            return carry

        lax.fori_loop(0, SCCH // 16, fill, 0)
        pltpu.sync_copy(m_v, acc.at[loc_v], add=True)
    plsc.subcore_barrier()
    pltpu.sync_copy(acc.at[pl.ds(s * OROWS, OROWS)],
                    out_hbm.at[pl.ds(c * HALF + s * OROWS, OROWS)])


_scatter_add = functools.partial(
    pl.kernel,
    mesh=_MESH,
    out_type=jax.ShapeDtypeStruct((2 * HALF, F * L2), jnp.float32),
    scratch_types=[
        pltpu.VMEM((SCCH,), jnp.int32),
        pltpu.VMEM((SCCH,), jnp.int32),
        pltpu.VMEM((SCCH, F * L2), jnp.float32),
        pltpu.VMEM_SHARED((ACC_ROWS, F * L2), jnp.float32),
        pltpu.SemaphoreType.DMA,
    ],
    compiler_params=_SC_PARAMS,
)(_scatter_body)


def _edge_messages_body(pos_s_ref, pos_r_ref, h_ref, w_r1_ref, w_r2_ref, m_ref):
    psT = jnp.transpose(pos_s_ref[...])
    prT = jnp.transpose(pos_r_ref[...])
    dx = prT[0] - psT[0]
    dy = prT[1] - psT[1]
    dz = prT[2] - psT[2]
    r2 = dx * dx + dy * dy + dz * dz
    r = jnp.sqrt(r2) + 1e-9
    inv_r = 1.0 / r
    x, y, z = dx * inv_r, dy * inv_r, dz * inv_r
    c1 = jnp.sqrt(3.0)
    c2 = jnp.sqrt(15.0)
    one = jnp.ones_like(x)
    shT = jnp.stack([
        one,
        c1 * x, c1 * y, c1 * z,
        c2 * x * y,
        c2 * y * z,
        (jnp.sqrt(5.0) / 2.0) * (3.0 * z * z - 1.0),
        c2 * x * z,
        (c2 / 2.0) * (x * x - y * y),
    ], axis=0)

    theta = (jnp.pi / R_MAX) * r
    s1 = jnp.sin(theta)
    c1t = jnp.cos(theta)
    two_c = 2.0 * c1t
    sins = [s1, two_c * s1]
    for _ in range(2, NB):
        sins.append(two_c * sins[-1] - sins[-2])
    p = 5.0
    xr = r / R_MAX
    xp = xr ** 5
    env = (1.0 - ((p + 1.0) * (p + 2.0) / 2.0) * xp
           + p * (p + 2.0) * xp * xr
           - (p * (p + 1.0) / 2.0) * xp * xr * xr)
    env = jnp.where(xr < 1.0, env, 0.0)
    scale = jnp.sqrt(2.0 / R_MAX) * inv_r * env
    efT = jnp.stack([s * scale for s in sins], axis=0)

    sh = jnp.transpose(shT)
    ef = jnp.transpose(efT)
    pre = jnp.dot(ef, w_r1_ref[...], preferred_element_type=jnp.float32)
    zact = pre * jax.nn.sigmoid(pre)
    tpw = jnp.dot(zact, w_r2_ref[...], preferred_element_type=jnp.float32)
    h = h_ref[...]
    col = lax.broadcasted_iota(jnp.int32, (F, F * L2), 1)
    row = lax.broadcasted_iota(jnp.int32, (F, F * L2), 0)
    rep = (col // L2 == row).astype(jnp.float32)
    col9 = lax.broadcasted_iota(jnp.int32, (L2, F * L2), 1)
    row9 = lax.broadcasted_iota(jnp.int32, (L2, F * L2), 0)
    til = (col9 % L2 == row9).astype(jnp.float32)
    h_rep = jnp.dot(h, rep, preferred_element_type=jnp.float32)
    sh_til = jnp.dot(sh, til, preferred_element_type=jnp.float32)
    m_ref[...] = h_rep * sh_til * tpw


def _edge_messages(pos_s, pos_r, h_gather, w_r1, w_r2):
    grid = (E_PAD // EB,)
    return pl.pallas_call(
        _edge_messages_body,
        grid=grid,
        in_specs=[
            pl.BlockSpec((EB, 16), lambda i: (i, 0)),
            pl.BlockSpec((EB, 16), lambda i: (i, 0)),
            pl.BlockSpec((EB, F), lambda i: (i, 0)),
            pl.BlockSpec((NB, F), lambda i: (0, 0)),
            pl.BlockSpec((F, F * L2), lambda i: (0, 0)),
        ],
        out_specs=pl.BlockSpec((EB, F * L2), lambda i: (i, 0)),
        out_shape=jax.ShapeDtypeStruct((E_PAD, F * L2), jnp.float32),
    )(pos_s, pos_r, h_gather, w_r1, w_r2)


def kernel(positions, node_attrs, shifts, eps, w_embed, w_r1_0, w_r2_0, w_r1_1, w_r2_1,
           w_sc_0, w_sc_1, w_read_0, w_read_1, edge_index, batch):
    alphas = 1.0 - jnp.linspace(1e-4, 0.02, T)
    abar = jnp.cumprod(alphas)[T_IDX]
    s = jnp.sqrt(abar)
    sq = jnp.sqrt(1.0 - abar)
    node_attrs = node_attrs / 4.0
    pos_n = s * positions + sq * eps[:, -3:]
    attrs_n = s * node_attrs + sq * eps[:, :NE]
    t_feat = jnp.full((node_attrs.shape[0], 1), T_IDX / float(T), dtype=jnp.float32)
    h0 = jnp.concatenate([attrs_n, t_feat], axis=-1) @ w_embed
    node_feats = jnp.zeros((N, F, L2), dtype=jnp.float32).at[:, :, 0].set(h0)

    sender = edge_index[0]
    receiver = edge_index[1]
    pad = E_PAD - E
    snd_pad = jnp.concatenate([sender, jnp.zeros((pad,), jnp.int32)])
    rcv_pad = jnp.concatenate([receiver, jnp.zeros((pad,), jnp.int32)])
    rcv_scatter = jnp.concatenate([receiver, jnp.full((pad,), N, jnp.int32)])

    pos_tbl = jnp.zeros((N, 16), jnp.float32).at[:, :3].set(pos_n)
    pos_s16, pos_r16 = _gather_geom(pos_tbl, snd_pad, rcv_pad)

    preds_sum = jnp.zeros((N, NOUT), dtype=jnp.float32)
    for w_r1, w_r2, w_sc, w_read in ((w_r1_0, w_r2_0, w_sc_0, w_read_0),
                                     (w_r1_1, w_r2_1, w_sc_1, w_read_1)):
        h_tbl = node_feats[:, :, 0]
        h_gather = _gather_h(h_tbl, snd_pad)
        m = _edge_messages(pos_s16, pos_r16, h_gather, w_r1, w_r2)
        zeros_acc = jnp.zeros((ACC_ROWS, F * L2), jnp.float32)
        agg = _scatter_add(m, rcv_scatter, zeros_acc)[:N] / AVG_NEI
        agg = agg.reshape(N, F, L2)
        sc = jnp.einsum("nfl,fg->ngl", node_feats, w_sc)
        nf = agg + sc
        inv = jnp.sum(nf ** 2, axis=-1)
        nf = nf * (1.0 + 0.1 * jnp.tanh(inv))[:, :, None]
        node_feats = nf
        preds_sum = preds_sum + nf.reshape(N, F * L2) @ w_read

    pn_pos = preds_sum[:, -3:]
    pn_lab = preds_sum[:, :-3]
    err_pos = (pn_pos - eps[:, -3:]) ** 2
    err_lab = (pn_lab - eps[:, :NE]) ** 2
    lp = jax.ops.segment_sum(err_pos, batch, num_segments=1).sum(axis=-1)
    ll = jax.ops.segment_sum(err_lab, batch, num_segments=1).sum(axis=-1)
    num_nodes = jnp.array([float(N)], dtype=jnp.float32)
    loss = 0.5 * (lp + ll) / (num_nodes * (3.0 + NE))
    return (pn_lab, pn_pos, eps[:, :NE], eps[:, -3:], loss)


# speedup vs baseline: 15.0294x; 1.0511x over previous
import functools

import jax
import jax.numpy as jnp
from jax import lax
from jax.experimental import pallas as pl
from jax.experimental.pallas import tpu as pltpu
from jax.experimental.pallas import tpu_sc as plsc

N = 10000
E = 160000
NE = 4
F = 32
L2 = 9
NB = 8
T = 1000
R_MAX = 5.0
AVG_NEI = 16.0
NOUT = NE + 3
T_IDX = 500

NC = 2
NS = 16
NW = NC * NS
CH = 128
E_PAD = 163840
EPW = E_PAD // NW
NCHUNK = EPW // CH

EB = 1024

_MESH = plsc.VectorSubcoreMesh(core_axis_name="c", subcore_axis_name="s")
_SC_PARAMS = pltpu.CompilerParams(use_tc_tiling_on_sc=False)


def _gather_geom_body(pos_tbl, snd, rcv, out_s, out_r, idx_s, idx_r, rows_s, rows_r, sem):
    wid = lax.axis_index("s") * NC + lax.axis_index("c")
    base = wid * EPW
    pltpu.sync_copy(snd.at[pl.ds(base, EPW)], idx_s)
    pltpu.sync_copy(rcv.at[pl.ds(base, EPW)], idx_r)
    for j in range(NCHUNK):
        pltpu.async_copy(pos_tbl.at[idx_s.at[pl.ds(j * CH, CH)]], rows_s, sem).wait()
        pltpu.sync_copy(rows_s, out_s.at[pl.ds(base + j * CH, CH)])
        pltpu.async_copy(pos_tbl.at[idx_r.at[pl.ds(j * CH, CH)]], rows_r, sem).wait()
        pltpu.sync_copy(rows_r, out_r.at[pl.ds(base + j * CH, CH)])


_gather_geom = functools.partial(
    pl.kernel,
    mesh=_MESH,
    out_type=[
        jax.ShapeDtypeStruct((E_PAD, 16), jnp.float32),
        jax.ShapeDtypeStruct((E_PAD, 16), jnp.float32),
    ],
    scratch_types=[
        pltpu.VMEM((EPW,), jnp.int32),
        pltpu.VMEM((EPW,), jnp.int32),
        pltpu.VMEM((CH, 16), jnp.float32),
        pltpu.VMEM((CH, 16), jnp.float32),
        pltpu.SemaphoreType.DMA,
    ],
    compiler_params=_SC_PARAMS,
)(_gather_geom_body)


def _gather_h_body(h_tbl, snd, out_h, idx_s, rows_h, sem):
    wid = lax.axis_index("s") * NC + lax.axis_index("c")
    base = wid * EPW
    pltpu.sync_copy(snd.at[pl.ds(base, EPW)], idx_s)
    for j in range(NCHUNK):
        pltpu.async_copy(h_tbl.at[idx_s.at[pl.ds(j * CH, CH)]], rows_h, sem).wait()
        pltpu.sync_copy(rows_h, out_h.at[pl.ds(base + j * CH, CH)])


_gather_h = functools.partial(
    pl.kernel,
    mesh=_MESH,
    out_type=jax.ShapeDtypeStruct((E_PAD, F), jnp.float32),
    scratch_types=[
        pltpu.VMEM((EPW,), jnp.int32),
        pltpu.VMEM((CH, F), jnp.float32),
        pltpu.SemaphoreType.DMA,
    ],
    compiler_params=_SC_PARAMS,
)(_gather_h_body)


HALF = 5120
TRASH = 256
ACC_ROWS = HALF + TRASH
ES = E_PAD // NS
SCH = ES // CH
ZROWS = ACC_ROWS // NS
OROWS = HALF // NS


SCCH = 80
SSCH = ES // SCCH


def _scatter_body(m_hbm, rcv_hbm, zeros_hbm, out_hbm, ridx_v, loc_v, m_v, acc, sem):
    c = lax.axis_index("c")
    s = lax.axis_index("s")
    pltpu.sync_copy(zeros_hbm.at[pl.ds(s * ZROWS, ZROWS)], acc.at[pl.ds(s * ZROWS, ZROWS)])
    base = s * ES
    half_base = c * HALF
    iota16 = lax.broadcasted_iota(jnp.int32, (16,), 0)
    plsc.subcore_barrier()
    for j in range(SSCH):
        pltpu.sync_copy(rcv_hbm.at[pl.ds(base + j * SCCH, SCCH)], ridx_v)
        pltpu.sync_copy(m_hbm.at[pl.ds(base + j * SCCH, SCCH)], m_v)

        def fill(g, carry, j=j):
            iv = ridx_v[pl.ds(g * 16, 16)]
            loc = iv - half_base
            own = (loc >= 0) & (loc < HALF)
            trash = HALF + (((j * SCCH + g * 16) + iota16) & (TRASH - 1))
            loc_v[pl.ds(g * 16, 16)] = jnp.where(own, loc, trash)
            return carry

        lax.fori_loop(0, SCCH // 16, fill, 0)
        pltpu.sync_copy(m_v, acc.at[loc_v], add=True)
    plsc.subcore_barrier()
    pltpu.sync_copy(acc.at[pl.ds(s * OROWS, OROWS)],
                    out_hbm.at[pl.ds(c * HALF + s * OROWS, OROWS)])


_scatter_add = functools.partial(
    pl.kernel,
    mesh=_MESH,
    out_type=jax.ShapeDtypeStruct((2 * HALF, F * L2), jnp.float32),
    scratch_types=[
        pltpu.VMEM((SCCH,), jnp.int32),
        pltpu.VMEM((SCCH,), jnp.int32),
        pltpu.VMEM((SCCH, F * L2), jnp.float32),
        pltpu.VMEM_SHARED((ACC_ROWS, F * L2), jnp.float32),
        pltpu.SemaphoreType.DMA,
    ],
    compiler_params=_SC_PARAMS,
)(_scatter_body)


def _edge_messages_body(pos_s_ref, pos_r_ref, h_ref, w_r1_ref, w_r2_ref, m_ref):
    psT = jnp.transpose(pos_s_ref[...])
    prT = jnp.transpose(pos_r_ref[...])
    dx = prT[0] - psT[0]
    dy = prT[1] - psT[1]
    dz = prT[2] - psT[2]
    r2 = dx * dx + dy * dy + dz * dz
    r = jnp.sqrt(r2) + 1e-9
    inv_r = 1.0 / r
    x, y, z = dx * inv_r, dy * inv_r, dz * inv_r
    c1 = jnp.sqrt(3.0)
    c2 = jnp.sqrt(15.0)
    one = jnp.ones_like(x)
    shT = jnp.stack([
        one,
        c1 * x, c1 * y, c1 * z,
        c2 * x * y,
        c2 * y * z,
        (jnp.sqrt(5.0) / 2.0) * (3.0 * z * z - 1.0),
        c2 * x * z,
        (c2 / 2.0) * (x * x - y * y),
    ], axis=0)

    theta = (jnp.pi / R_MAX) * r
    s1 = jnp.sin(theta)
    c1t = jnp.cos(theta)
    two_c = 2.0 * c1t
    sins = [s1, two_c * s1]
    for _ in range(2, NB):
        sins.append(two_c * sins[-1] - sins[-2])
    p = 5.0
    xr = r / R_MAX
    xp = xr ** 5
    env = (1.0 - ((p + 1.0) * (p + 2.0) / 2.0) * xp
           + p * (p + 2.0) * xp * xr
           - (p * (p + 1.0) / 2.0) * xp * xr * xr)
    env = jnp.where(xr < 1.0, env, 0.0)
    scale = jnp.sqrt(2.0 / R_MAX) * inv_r * env
    efT = jnp.stack([s * scale for s in sins], axis=0)

    sh = jnp.transpose(shT)
    ef = jnp.transpose(efT)
    pre = jnp.dot(ef, w_r1_ref[...], preferred_element_type=jnp.float32)
    zact = pre * jax.nn.sigmoid(pre)
    tpw = jnp.dot(zact, w_r2_ref[...], preferred_element_type=jnp.float32)
    h = h_ref[...]
    col = lax.broadcasted_iota(jnp.int32, (F, F * L2), 1)
    row = lax.broadcasted_iota(jnp.int32, (F, F * L2), 0)
    rep = (col // L2 == row).astype(jnp.float32)
    col9 = lax.broadcasted_iota(jnp.int32, (L2, F * L2), 1)
    row9 = lax.broadcasted_iota(jnp.int32, (L2, F * L2), 0)
    til = (col9 % L2 == row9).astype(jnp.float32)
    h_rep = jnp.dot(h, rep, preferred_element_type=jnp.float32)
    sh_til = jnp.dot(sh, til, preferred_element_type=jnp.float32)
    m_ref[...] = h_rep * sh_til * tpw


def _edge_messages(pos_s, pos_r, h_gather, w_r1, w_r2):
    grid = (E_PAD // EB,)
    return pl.pallas_call(
        _edge_messages_body,
        grid=grid,
        in_specs=[
            pl.BlockSpec((EB, 16), lambda i: (i, 0)),
            pl.BlockSpec((EB, 16), lambda i: (i, 0)),
            pl.BlockSpec((EB, F), lambda i: (i, 0)),
            pl.BlockSpec((NB, F), lambda i: (0, 0)),
            pl.BlockSpec((F, F * L2), lambda i: (0, 0)),
        ],
        out_specs=pl.BlockSpec((EB, F * L2), lambda i: (i, 0)),
        out_shape=jax.ShapeDtypeStruct((E_PAD, F * L2), jnp.float32),
    )(pos_s, pos_r, h_gather, w_r1, w_r2)


N_PAD = 2 * HALF
NBK = 1024


def _h0_body(xcat_ref, w_embed_ref, h0_ref):
    h0_ref[...] = jnp.dot(xcat_ref[...], w_embed_ref[...], preferred_element_type=jnp.float32)


def _h0_embed(xcat, w_embed):
    return pl.pallas_call(
        _h0_body,
        grid=(N_PAD // NBK,),
        in_specs=[
            pl.BlockSpec((NBK, NE + 1), lambda i: (i, 0)),
            pl.BlockSpec((NE + 1, F), lambda i: (0, 0)),
        ],
        out_specs=pl.BlockSpec((NBK, F), lambda i: (i, 0)),
        out_shape=jax.ShapeDtypeStruct((N_PAD, F), jnp.float32),
    )(xcat, w_embed)


def _node_update_body(agg_ref, h_ref, w_big_ref, g_ref, gl0_ref, rep_ref, w_read_ref,
                      nf_ref, l0_ref, preds_ref):
    hi = jax.lax.Precision.HIGHEST
    agg = agg_ref[...] * (1.0 / AVG_NEI)
    sc = jnp.dot(h_ref[...], w_big_ref[...], precision=hi, preferred_element_type=jnp.float32)
    nf = agg + sc
    inv = jnp.dot(nf * nf, g_ref[...], precision=hi, preferred_element_type=jnp.float32)
    gate = 1.0 + 0.1 * jnp.tanh(inv)
    gbig = jnp.dot(gate, rep_ref[...], precision=hi, preferred_element_type=jnp.float32)
    nfn = nf * gbig
    nf_ref[...] = nfn
    l0_ref[...] = jnp.dot(nfn, gl0_ref[...], precision=hi, preferred_element_type=jnp.float32)
    preds_ref[...] = jnp.dot(nfn, w_read_ref[...], precision=hi, preferred_element_type=jnp.float32)


def _node_update(agg_raw, h_in, w_big, g_mat, gl0_mat, rep_mat, w_read):
    kin = h_in.shape[1]
    return pl.pallas_call(
        _node_update_body,
        grid=(N_PAD // NBK,),
        in_specs=[
            pl.BlockSpec((NBK, F * L2), lambda i: (i, 0)),
            pl.BlockSpec((NBK, kin), lambda i: (i, 0)),
            pl.BlockSpec((kin, F * L2), lambda i: (0, 0)),
            pl.BlockSpec((F * L2, F), lambda i: (0, 0)),
            pl.BlockSpec((F * L2, F), lambda i: (0, 0)),
            pl.BlockSpec((F, F * L2), lambda i: (0, 0)),
            pl.BlockSpec((F * L2, NOUT), lambda i: (0, 0)),
        ],
        out_specs=[
            pl.BlockSpec((NBK, F * L2), lambda i: (i, 0)),
            pl.BlockSpec((NBK, F), lambda i: (i, 0)),
            pl.BlockSpec((NBK, NOUT), lambda i: (i, 0)),
        ],
        out_shape=[
            jax.ShapeDtypeStruct((N_PAD, F * L2), jnp.float32),
            jax.ShapeDtypeStruct((N_PAD, F), jnp.float32),
            jax.ShapeDtypeStruct((N_PAD, NOUT), jnp.float32),
        ],
    )(agg_raw, h_in, w_big, g_mat, gl0_mat, rep_mat, w_read)


def kernel(positions, node_attrs, shifts, eps, w_embed, w_r1_0, w_r2_0, w_r1_1, w_r2_1,
           w_sc_0, w_sc_1, w_read_0, w_read_1, edge_index, batch):
    alphas = 1.0 - jnp.linspace(1e-4, 0.02, T)
    abar = jnp.cumprod(alphas)[T_IDX]
    s = jnp.sqrt(abar)
    sq = jnp.sqrt(1.0 - abar)
    node_attrs = node_attrs / 4.0
    pos_n = s * positions + sq * eps[:, -3:]
    attrs_n = s * node_attrs + sq * eps[:, :NE]
    t_feat = jnp.full((N, 1), T_IDX / float(T), dtype=jnp.float32)
    xcat = jnp.concatenate([attrs_n, t_feat], axis=-1)
    xcat_pad = jnp.zeros((N_PAD, NE + 1), jnp.float32).at[:N].set(xcat)
    h0_tbl = _h0_embed(xcat_pad, w_embed)

    sender = edge_index[0]
    receiver = edge_index[1]
    pad = E_PAD - E
    snd_pad = jnp.concatenate([sender, jnp.zeros((pad,), jnp.int32)])
    rcv_pad = jnp.concatenate([receiver, jnp.zeros((pad,), jnp.int32)])
    rcv_scatter = jnp.concatenate([receiver, jnp.full((pad,), N, jnp.int32)])

    pos_tbl = jnp.zeros((N, 16), jnp.float32).at[:, :3].set(pos_n)
    pos_s16, pos_r16 = _gather_geom(pos_tbl, snd_pad, rcv_pad)

    eye_f = jnp.eye(F, dtype=jnp.float32)
    eye_l = jnp.eye(L2, dtype=jnp.float32)
    rep_mat = jnp.kron(eye_f, jnp.ones((1, L2), jnp.float32))
    g_mat = rep_mat.T
    e0 = jnp.zeros((1, L2), jnp.float32).at[0, 0].set(1.0)
    gl0_mat = jnp.kron(eye_f, e0).T
    w_big_1 = jnp.kron(w_sc_1, eye_l)
    w0_big = jnp.kron(w_sc_0, e0)
    zeros_acc = jnp.zeros((ACC_ROWS, F * L2), jnp.float32)

    h_gather = _gather_h(h0_tbl, snd_pad)
    m = _edge_messages(pos_s16, pos_r16, h_gather, w_r1_0, w_r2_0)
    agg_raw = _scatter_add(m, rcv_scatter, zeros_acc)
    nf1, l0_1, preds0 = _node_update(agg_raw, h0_tbl, w0_big, g_mat, gl0_mat, rep_mat, w_read_0)

    h_gather = _gather_h(l0_1, snd_pad)
    m = _edge_messages(pos_s16, pos_r16, h_gather, w_r1_1, w_r2_1)
    agg_raw = _scatter_add(m, rcv_scatter, zeros_acc)
    _, _, preds1 = _node_update(agg_raw, nf1, w_big_1, g_mat, gl0_mat, rep_mat, w_read_1)

    preds_sum = (preds0 + preds1)[:N]

    pn_pos = preds_sum[:, -3:]
    pn_lab = preds_sum[:, :-3]
    err_pos = (pn_pos - eps[:, -3:]) ** 2
    err_lab = (pn_lab - eps[:, :NE]) ** 2
    lp = jax.ops.segment_sum(err_pos, batch, num_segments=1).sum(axis=-1)
    ll = jax.ops.segment_sum(err_lab, batch, num_segments=1).sum(axis=-1)
    num_nodes = jnp.array([float(N)], dtype=jnp.float32)
    loss = 0.5 * (lp + ll) / (num_nodes * (3.0 + NE))
    return (pn_lab, pn_pos, eps[:, :NE], eps[:, -3:], loss)


# speedup vs baseline: 15.7634x; 1.0488x over previous
import functools

import jax
import jax.numpy as jnp
from jax import lax
from jax.experimental import pallas as pl
from jax.experimental.pallas import tpu as pltpu
from jax.experimental.pallas import tpu_sc as plsc

N = 10000
E = 160000
NE = 4
F = 32
L2 = 9
NB = 8
T = 1000
R_MAX = 5.0
AVG_NEI = 16.0
NOUT = NE + 3
T_IDX = 500

NC = 2
NS = 16
NW = NC * NS
CH = 128
E_PAD = 163840
EPW = E_PAD // NW
NCHUNK = EPW // CH

EB = 1024

_MESH = plsc.VectorSubcoreMesh(core_axis_name="c", subcore_axis_name="s")
_SC_PARAMS = pltpu.CompilerParams(use_tc_tiling_on_sc=False)


def _gather_geom_body(pos_tbl, snd, rcv, out_s, out_r, idx_s, idx_r, rows_s, rows_r, sem):
    wid = lax.axis_index("s") * NC + lax.axis_index("c")
    base = wid * EPW
    pltpu.sync_copy(snd.at[pl.ds(base, EPW)], idx_s)
    pltpu.sync_copy(rcv.at[pl.ds(base, EPW)], idx_r)
    for j in range(NCHUNK):
        pltpu.async_copy(pos_tbl.at[idx_s.at[pl.ds(j * CH, CH)]], rows_s, sem).wait()
        pltpu.sync_copy(rows_s, out_s.at[pl.ds(base + j * CH, CH)])
        pltpu.async_copy(pos_tbl.at[idx_r.at[pl.ds(j * CH, CH)]], rows_r, sem).wait()
        pltpu.sync_copy(rows_r, out_r.at[pl.ds(base + j * CH, CH)])


_gather_geom = functools.partial(
    pl.kernel,
    mesh=_MESH,
    out_type=[
        jax.ShapeDtypeStruct((E_PAD, 16), jnp.float32),
        jax.ShapeDtypeStruct((E_PAD, 16), jnp.float32),
    ],
    scratch_types=[
        pltpu.VMEM((EPW,), jnp.int32),
        pltpu.VMEM((EPW,), jnp.int32),
        pltpu.VMEM((CH, 16), jnp.float32),
        pltpu.VMEM((CH, 16), jnp.float32),
        pltpu.SemaphoreType.DMA,
    ],
    compiler_params=_SC_PARAMS,
)(_gather_geom_body)


def _gather_h_body(h_tbl, snd, out_h, idx_s, rows_h, sem):
    wid = lax.axis_index("s") * NC + lax.axis_index("c")
    base = wid * EPW
    pltpu.sync_copy(snd.at[pl.ds(base, EPW)], idx_s)
    for j in range(NCHUNK):
        pltpu.async_copy(h_tbl.at[idx_s.at[pl.ds(j * CH, CH)]], rows_h, sem).wait()
        pltpu.sync_copy(rows_h, out_h.at[pl.ds(base + j * CH, CH)])


_gather_h = functools.partial(
    pl.kernel,
    mesh=_MESH,
    out_type=jax.ShapeDtypeStruct((E_PAD, F), jnp.float32),
    scratch_types=[
        pltpu.VMEM((EPW,), jnp.int32),
        pltpu.VMEM((CH, F), jnp.float32),
        pltpu.SemaphoreType.DMA,
    ],
    compiler_params=_SC_PARAMS,
)(_gather_h_body)


HALF = 5120
TRASH = 256
ACC_ROWS = HALF + TRASH
ES = E_PAD // NS
SCH = ES // CH
ZROWS = ACC_ROWS // NS
OROWS = HALF // NS


SCCH = 80
SSCH = ES // SCCH


def _scatter_body(m_hbm, rcv_hbm, zeros_hbm, out_hbm, ridx_v, loc_v, m_v, acc, sem):
    c = lax.axis_index("c")
    s = lax.axis_index("s")
    pltpu.sync_copy(zeros_hbm.at[pl.ds(s * ZROWS, ZROWS)], acc.at[pl.ds(s * ZROWS, ZROWS)])
    base = s * ES
    half_base = c * HALF
    iota16 = lax.broadcasted_iota(jnp.int32, (16,), 0)
    plsc.subcore_barrier()
    for j in range(SSCH):
        pltpu.sync_copy(rcv_hbm.at[pl.ds(base + j * SCCH, SCCH)], ridx_v)
        pltpu.sync_copy(m_hbm.at[pl.ds(base + j * SCCH, SCCH)], m_v)

        def fill(g, carry, j=j):
            iv = ridx_v[pl.ds(g * 16, 16)]
            loc = iv - half_base
            own = (loc >= 0) & (loc < HALF)
            trash = HALF + (((j * SCCH + g * 16) + iota16) & (TRASH - 1))
            loc_v[pl.ds(g * 16, 16)] = jnp.where(own, loc, trash)
            return carry

        lax.fori_loop(0, SCCH // 16, fill, 0)
        pltpu.sync_copy(m_v, acc.at[loc_v], add=True)
    plsc.subcore_barrier()
    pltpu.sync_copy(acc.at[pl.ds(s * OROWS, OROWS)],
                    out_hbm.at[pl.ds(c * HALF + s * OROWS, OROWS)])


_scatter_add = functools.partial(
    pl.kernel,
    mesh=_MESH,
    out_type=jax.ShapeDtypeStruct((2 * HALF, F * L2), jnp.float32),
    scratch_types=[
        pltpu.VMEM((SCCH,), jnp.int32),
        pltpu.VMEM((SCCH,), jnp.int32),
        pltpu.VMEM((SCCH, F * L2), jnp.float32),
        pltpu.VMEM_SHARED((ACC_ROWS, F * L2), jnp.float32),
        pltpu.SemaphoreType.DMA,
    ],
    compiler_params=_SC_PARAMS,
)(_scatter_body)


def _edge_messages_body(pos_s_ref, pos_r_ref, h_ref, w_r1_ref, w_r2_ref, m_ref):
    psT = jnp.transpose(pos_s_ref[...])
    prT = jnp.transpose(pos_r_ref[...])
    dx = prT[0] - psT[0]
    dy = prT[1] - psT[1]
    dz = prT[2] - psT[2]
    r2 = dx * dx + dy * dy + dz * dz
    r = jnp.sqrt(r2) + 1e-9
    inv_r = 1.0 / r
    x, y, z = dx * inv_r, dy * inv_r, dz * inv_r
    c1 = jnp.sqrt(3.0)
    c2 = jnp.sqrt(15.0)
    one = jnp.ones_like(x)
    shT = jnp.stack([
        one,
        c1 * x, c1 * y, c1 * z,
        c2 * x * y,
        c2 * y * z,
        (jnp.sqrt(5.0) / 2.0) * (3.0 * z * z - 1.0),
        c2 * x * z,
        (c2 / 2.0) * (x * x - y * y),
    ], axis=0)

    theta = (jnp.pi / R_MAX) * r
    s1 = jnp.sin(theta)
    c1t = jnp.cos(theta)
    two_c = 2.0 * c1t
    sins = [s1, two_c * s1]
    for _ in range(2, NB):
        sins.append(two_c * sins[-1] - sins[-2])
    p = 5.0
    xr = r / R_MAX
    xp = xr ** 5
    env = (1.0 - ((p + 1.0) * (p + 2.0) / 2.0) * xp
           + p * (p + 2.0) * xp * xr
           - (p * (p + 1.0) / 2.0) * xp * xr * xr)
    env = jnp.where(xr < 1.0, env, 0.0)
    scale = jnp.sqrt(2.0 / R_MAX) * inv_r * env
    efT = jnp.stack([s * scale for s in sins], axis=0)

    sh = jnp.transpose(shT)
    ef = jnp.transpose(efT)
    pre = jnp.dot(ef, w_r1_ref[...], preferred_element_type=jnp.float32)
    zact = pre * jax.nn.sigmoid(pre)
    tpw = jnp.dot(zact, w_r2_ref[...], preferred_element_type=jnp.float32)
    h = h_ref[...]
    col = lax.broadcasted_iota(jnp.int32, (F, F * L2), 1)
    row = lax.broadcasted_iota(jnp.int32, (F, F * L2), 0)
    rep = (col // L2 == row).astype(jnp.float32)
    col9 = lax.broadcasted_iota(jnp.int32, (L2, F * L2), 1)
    row9 = lax.broadcasted_iota(jnp.int32, (L2, F * L2), 0)
    til = (col9 % L2 == row9).astype(jnp.float32)
    h_rep = jnp.dot(h, rep, preferred_element_type=jnp.float32)
    sh_til = jnp.dot(sh, til, preferred_element_type=jnp.float32)
    m_ref[...] = h_rep * sh_til * tpw


def _edge_messages(pos_s, pos_r, h_gather, w_r1, w_r2):
    grid = (E_PAD // EB,)
    return pl.pallas_call(
        _edge_messages_body,
        grid=grid,
        in_specs=[
            pl.BlockSpec((EB, 16), lambda i: (i, 0)),
            pl.BlockSpec((EB, 16), lambda i: (i, 0)),
            pl.BlockSpec((EB, F), lambda i: (i, 0)),
            pl.BlockSpec((NB, F), lambda i: (0, 0)),
            pl.BlockSpec((F, F * L2), lambda i: (0, 0)),
        ],
        out_specs=pl.BlockSpec((EB, F * L2), lambda i: (i, 0)),
        out_shape=jax.ShapeDtypeStruct((E_PAD, F * L2), jnp.float32),
    )(pos_s, pos_r, h_gather, w_r1, w_r2)


N_PAD = 2 * HALF
NBK = 1024


def _h0_body(xcat_ref, w_embed_ref, h0_ref):
    h0_ref[...] = jnp.dot(xcat_ref[...], w_embed_ref[...], preferred_element_type=jnp.float32)


def _h0_embed(xcat, w_embed):
    return pl.pallas_call(
        _h0_body,
        grid=(N_PAD // NBK,),
        in_specs=[
            pl.BlockSpec((NBK, NE + 1), lambda i: (i, 0)),
            pl.BlockSpec((NE + 1, F), lambda i: (0, 0)),
        ],
        out_specs=pl.BlockSpec((NBK, F), lambda i: (i, 0)),
        out_shape=jax.ShapeDtypeStruct((N_PAD, F), jnp.float32),
    )(xcat, w_embed)


def _node_update_body(agg_ref, h_ref, w_big_ref, g_ref, gl0_ref, rep_ref, w_read_ref,
                      nf_ref, l0_ref, preds_ref):
    agg = agg_ref[...] * (1.0 / AVG_NEI)
    sc = jnp.dot(h_ref[...], w_big_ref[...], preferred_element_type=jnp.float32)
    nf = agg + sc
    inv = jnp.dot(nf * nf, g_ref[...], preferred_element_type=jnp.float32)
    en = jnp.exp(-2.0 * inv)
    gate = 1.0 + 0.1 * ((1.0 - en) / (1.0 + en))
    gbig = jnp.dot(gate, rep_ref[...], preferred_element_type=jnp.float32)
    nfn = nf * gbig
    nf_ref[...] = nfn
    l0_ref[...] = jnp.dot(nfn, gl0_ref[...], preferred_element_type=jnp.float32)
    preds_ref[...] = jnp.dot(nfn, w_read_ref[...], preferred_element_type=jnp.float32)


def _node_update(agg_raw, h_in, w_big, g_mat, gl0_mat, rep_mat, w_read):
    kin = h_in.shape[1]
    return pl.pallas_call(
        _node_update_body,
        grid=(N_PAD // NBK,),
        in_specs=[
            pl.BlockSpec((NBK, F * L2), lambda i: (i, 0)),
            pl.BlockSpec((NBK, kin), lambda i: (i, 0)),
            pl.BlockSpec((kin, F * L2), lambda i: (0, 0)),
            pl.BlockSpec((F * L2, F), lambda i: (0, 0)),
            pl.BlockSpec((F * L2, F), lambda i: (0, 0)),
            pl.BlockSpec((F, F * L2), lambda i: (0, 0)),
            pl.BlockSpec((F * L2, NOUT), lambda i: (0, 0)),
        ],
        out_specs=[
            pl.BlockSpec((NBK, F * L2), lambda i: (i, 0)),
            pl.BlockSpec((NBK, F), lambda i: (i, 0)),
            pl.BlockSpec((NBK, NOUT), lambda i: (i, 0)),
        ],
        out_shape=[
            jax.ShapeDtypeStruct((N_PAD, F * L2), jnp.float32),
            jax.ShapeDtypeStruct((N_PAD, F), jnp.float32),
            jax.ShapeDtypeStruct((N_PAD, NOUT), jnp.float32),
        ],
    )(agg_raw, h_in, w_big, g_mat, gl0_mat, rep_mat, w_read)


def kernel(positions, node_attrs, shifts, eps, w_embed, w_r1_0, w_r2_0, w_r1_1, w_r2_1,
           w_sc_0, w_sc_1, w_read_0, w_read_1, edge_index, batch):
    alphas = 1.0 - jnp.linspace(1e-4, 0.02, T)
    abar = jnp.cumprod(alphas)[T_IDX]
    s = jnp.sqrt(abar)
    sq = jnp.sqrt(1.0 - abar)
    node_attrs = node_attrs / 4.0
    pos_n = s * positions + sq * eps[:, -3:]
    attrs_n = s * node_attrs + sq * eps[:, :NE]
    t_feat = jnp.full((N, 1), T_IDX / float(T), dtype=jnp.float32)
    xcat = jnp.concatenate([attrs_n, t_feat], axis=-1)
    xcat_pad = jnp.zeros((N_PAD, NE + 1), jnp.float32).at[:N].set(xcat)
    h0_tbl = _h0_embed(xcat_pad, w_embed)

    sender = edge_index[0]
    receiver = edge_index[1]
    pad = E_PAD - E
    snd_pad = jnp.concatenate([sender, jnp.zeros((pad,), jnp.int32)])
    rcv_pad = jnp.concatenate([receiver, jnp.zeros((pad,), jnp.int32)])
    rcv_scatter = jnp.concatenate([receiver, jnp.full((pad,), N, jnp.int32)])

    pos_tbl = jnp.zeros((N, 16), jnp.float32).at[:, :3].set(pos_n)
    pos_s16, pos_r16 = _gather_geom(pos_tbl, snd_pad, rcv_pad)

    eye_f = jnp.eye(F, dtype=jnp.float32)
    eye_l = jnp.eye(L2, dtype=jnp.float32)
    rep_mat = jnp.kron(eye_f, jnp.ones((1, L2), jnp.float32))
    g_mat = rep_mat.T
    e0 = jnp.zeros((1, L2), jnp.float32).at[0, 0].set(1.0)
    gl0_mat = jnp.kron(eye_f, e0).T
    w_big_1 = jnp.kron(w_sc_1, eye_l)
    w0_big = jnp.kron(w_sc_0, e0)
    zeros_acc = jnp.zeros((ACC_ROWS, F * L2), jnp.float32)

    h_gather = _gather_h(h0_tbl, snd_pad)
    m = _edge_messages(pos_s16, pos_r16, h_gather, w_r1_0, w_r2_0)
    agg_raw = _scatter_add(m, rcv_scatter, zeros_acc)
    nf1, l0_1, preds0 = _node_update(agg_raw, h0_tbl, w0_big, g_mat, gl0_mat, rep_mat, w_read_0)

    h_gather = _gather_h(l0_1, snd_pad)
    m = _edge_messages(pos_s16, pos_r16, h_gather, w_r1_1, w_r2_1)
    agg_raw = _scatter_add(m, rcv_scatter, zeros_acc)
    _, _, preds1 = _node_update(agg_raw, nf1, w_big_1, g_mat, gl0_mat, rep_mat, w_read_1)

    preds_sum = (preds0 + preds1)[:N]

    pn_pos = preds_sum[:, -3:]
    pn_lab = preds_sum[:, :-3]
    err_pos = (pn_pos - eps[:, -3:]) ** 2
    err_lab = (pn_lab - eps[:, :NE]) ** 2
    lp = jax.ops.segment_sum(err_pos, batch, num_segments=1).sum(axis=-1)
    ll = jax.ops.segment_sum(err_lab, batch, num_segments=1).sum(axis=-1)
    num_nodes = jnp.array([float(N)], dtype=jnp.float32)
    loss = 0.5 * (lp + ll) / (num_nodes * (3.0 + NE))
    return (pn_lab, pn_pos, eps[:, :NE], eps[:, -3:], loss)


# speedup vs baseline: 16.3590x; 1.0378x over previous
import functools

import jax
import jax.numpy as jnp
from jax import lax
from jax.experimental import pallas as pl
from jax.experimental.pallas import tpu as pltpu
from jax.experimental.pallas import tpu_sc as plsc

N = 10000
E = 160000
NE = 4
F = 32
L2 = 9
NB = 8
T = 1000
R_MAX = 5.0
AVG_NEI = 16.0
NOUT = NE + 3
T_IDX = 500

NC = 2
NS = 16
NW = NC * NS
CH = 128
E_PAD = 163840
EPW = E_PAD // NW
NCHUNK = EPW // CH

EB = 1024

_MESH = plsc.VectorSubcoreMesh(core_axis_name="c", subcore_axis_name="s")
_SC_PARAMS = pltpu.CompilerParams(use_tc_tiling_on_sc=False)


def _gather_geom_body(pos_tbl, snd, rcv, out_s, out_r, idx_s, idx_r, rows_s, rows_r, sem):
    wid = lax.axis_index("s") * NC + lax.axis_index("c")
    base = wid * EPW
    pltpu.sync_copy(snd.at[pl.ds(base, EPW)], idx_s)
    pltpu.sync_copy(rcv.at[pl.ds(base, EPW)], idx_r)
    for j in range(NCHUNK):
        pltpu.async_copy(pos_tbl.at[idx_s.at[pl.ds(j * CH, CH)]], rows_s, sem).wait()
        pltpu.sync_copy(rows_s, out_s.at[pl.ds(base + j * CH, CH)])
        pltpu.async_copy(pos_tbl.at[idx_r.at[pl.ds(j * CH, CH)]], rows_r, sem).wait()
        pltpu.sync_copy(rows_r, out_r.at[pl.ds(base + j * CH, CH)])


_gather_geom = functools.partial(
    pl.kernel,
    mesh=_MESH,
    out_type=[
        jax.ShapeDtypeStruct((E_PAD, 16), jnp.float32),
        jax.ShapeDtypeStruct((E_PAD, 16), jnp.float32),
    ],
    scratch_types=[
        pltpu.VMEM((EPW,), jnp.int32),
        pltpu.VMEM((EPW,), jnp.int32),
        pltpu.VMEM((CH, 16), jnp.float32),
        pltpu.VMEM((CH, 16), jnp.float32),
        pltpu.SemaphoreType.DMA,
    ],
    compiler_params=_SC_PARAMS,
)(_gather_geom_body)


def _gather_h_body(h_tbl, snd, out_h, idx_s, rows_h, sem):
    wid = lax.axis_index("s") * NC + lax.axis_index("c")
    base = wid * EPW
    pltpu.sync_copy(snd.at[pl.ds(base, EPW)], idx_s)
    for j in range(NCHUNK):
        pltpu.async_copy(h_tbl.at[idx_s.at[pl.ds(j * CH, CH)]], rows_h, sem).wait()
        pltpu.sync_copy(rows_h, out_h.at[pl.ds(base + j * CH, CH)])


_gather_h = functools.partial(
    pl.kernel,
    mesh=_MESH,
    out_type=jax.ShapeDtypeStruct((E_PAD, F), jnp.float32),
    scratch_types=[
        pltpu.VMEM((EPW,), jnp.int32),
        pltpu.VMEM((CH, F), jnp.float32),
        pltpu.SemaphoreType.DMA,
    ],
    compiler_params=_SC_PARAMS,
)(_gather_h_body)


HALF = 5120
TRASH = 256
ACC_ROWS = HALF + TRASH
ES = E_PAD // NS
SCH = ES // CH
ZROWS = ACC_ROWS // NS
OROWS = HALF // NS


SCCH = 80
SSCH = ES // SCCH


def _scatter_body(m_hbm, rcv_hbm, zeros_hbm, out_hbm, ridx_v, loc_v, m_v, acc, sem):
    c = lax.axis_index("c")
    s = lax.axis_index("s")
    pltpu.sync_copy(zeros_hbm.at[pl.ds(s * ZROWS, ZROWS)], acc.at[pl.ds(s * ZROWS, ZROWS)])
    base = s * ES
    half_base = c * HALF
    iota16 = lax.broadcasted_iota(jnp.int32, (16,), 0)
    plsc.subcore_barrier()
    for j in range(SSCH):
        pltpu.sync_copy(rcv_hbm.at[pl.ds(base + j * SCCH, SCCH)], ridx_v)
        pltpu.sync_copy(m_hbm.at[pl.ds(base + j * SCCH, SCCH)], m_v)

        def fill(g, carry, j=j):
            iv = ridx_v[pl.ds(g * 16, 16)]
            loc = iv - half_base
            own = (loc >= 0) & (loc < HALF)
            trash = HALF + (((j * SCCH + g * 16) + iota16) & (TRASH - 1))
            loc_v[pl.ds(g * 16, 16)] = jnp.where(own, loc, trash)
            return carry

        lax.fori_loop(0, SCCH // 16, fill, 0)
        pltpu.sync_copy(m_v, acc.at[loc_v], add=True)
    plsc.subcore_barrier()
    pltpu.sync_copy(acc.at[pl.ds(s * OROWS, OROWS)],
                    out_hbm.at[pl.ds(c * HALF + s * OROWS, OROWS)])


_scatter_add = functools.partial(
    pl.kernel,
    mesh=_MESH,
    out_type=jax.ShapeDtypeStruct((2 * HALF, F * L2), jnp.float32),
    scratch_types=[
        pltpu.VMEM((SCCH,), jnp.int32),
        pltpu.VMEM((SCCH,), jnp.int32),
        pltpu.VMEM((SCCH, F * L2), jnp.float32),
        pltpu.VMEM_SHARED((ACC_ROWS, F * L2), jnp.float32),
        pltpu.SemaphoreType.DMA,
    ],
    compiler_params=_SC_PARAMS,
)(_scatter_body)


def _edge_messages_body(pos_s_ref, pos_r_ref, h_ref, w_r1_ref, w_r2_ref, m_ref):
    psT = jnp.transpose(pos_s_ref[...])
    prT = jnp.transpose(pos_r_ref[...])
    dx = prT[0] - psT[0]
    dy = prT[1] - psT[1]
    dz = prT[2] - psT[2]
    r2 = dx * dx + dy * dy + dz * dz
    r = jnp.sqrt(r2) + 1e-9
    inv_r = 1.0 / r
    x, y, z = dx * inv_r, dy * inv_r, dz * inv_r
    c1 = jnp.sqrt(3.0)
    c2 = jnp.sqrt(15.0)
    one = jnp.ones_like(x)
    shT = jnp.stack([
        one,
        c1 * x, c1 * y, c1 * z,
        c2 * x * y,
        c2 * y * z,
        (jnp.sqrt(5.0) / 2.0) * (3.0 * z * z - 1.0),
        c2 * x * z,
        (c2 / 2.0) * (x * x - y * y),
    ], axis=0)

    theta = (jnp.pi / R_MAX) * r
    s1 = jnp.sin(theta)
    c1t = jnp.cos(theta)
    two_c = 2.0 * c1t
    sins = [s1, two_c * s1]
    for _ in range(2, NB):
        sins.append(two_c * sins[-1] - sins[-2])
    p = 5.0
    xr = r / R_MAX
    xp = xr ** 5
    env = (1.0 - ((p + 1.0) * (p + 2.0) / 2.0) * xp
           + p * (p + 2.0) * xp * xr
           - (p * (p + 1.0) / 2.0) * xp * xr * xr)
    env = jnp.where(xr < 1.0, env, 0.0)
    scale = jnp.sqrt(2.0 / R_MAX) * inv_r * env
    efT = jnp.stack([s * scale for s in sins], axis=0)

    sh = jnp.transpose(shT)
    ef = jnp.transpose(efT)
    pre = jnp.dot(ef, w_r1_ref[...], preferred_element_type=jnp.float32)
    zact = pre * jax.nn.sigmoid(pre)
    tpw = jnp.dot(zact, w_r2_ref[...], preferred_element_type=jnp.float32)
    h = h_ref[...]
    col = lax.broadcasted_iota(jnp.int32, (F, F * L2), 1)
    row = lax.broadcasted_iota(jnp.int32, (F, F * L2), 0)
    rep = (col // L2 == row).astype(jnp.float32)
    col9 = lax.broadcasted_iota(jnp.int32, (L2, F * L2), 1)
    row9 = lax.broadcasted_iota(jnp.int32, (L2, F * L2), 0)
    til = (col9 % L2 == row9).astype(jnp.float32)
    h_rep = jnp.dot(h, rep, preferred_element_type=jnp.float32)
    sh_til = jnp.dot(sh, til, preferred_element_type=jnp.float32)
    m_ref[...] = h_rep * sh_til * tpw


def _edge_messages(pos_s, pos_r, h_gather, w_r1, w_r2):
    grid = (E_PAD // EB,)
    return pl.pallas_call(
        _edge_messages_body,
        grid=grid,
        in_specs=[
            pl.BlockSpec((EB, 16), lambda i: (i, 0)),
            pl.BlockSpec((EB, 16), lambda i: (i, 0)),
            pl.BlockSpec((EB, F), lambda i: (i, 0)),
            pl.BlockSpec((NB, F), lambda i: (0, 0)),
            pl.BlockSpec((F, F * L2), lambda i: (0, 0)),
        ],
        out_specs=pl.BlockSpec((EB, F * L2), lambda i: (i, 0)),
        out_shape=jax.ShapeDtypeStruct((E_PAD, F * L2), jnp.float32),
    )(pos_s, pos_r, h_gather, w_r1, w_r2)


N_PAD = 2 * HALF
NBK = 1024


def _h0_body(xcat_ref, w_embed_ref, h0_ref):
    h0_ref[...] = jnp.dot(xcat_ref[...], w_embed_ref[...], preferred_element_type=jnp.float32)


def _h0_embed(xcat, w_embed):
    return pl.pallas_call(
        _h0_body,
        grid=(N_PAD // NBK,),
        in_specs=[
            pl.BlockSpec((NBK, NE + 1), lambda i: (i, 0)),
            pl.BlockSpec((NE + 1, F), lambda i: (0, 0)),
        ],
        out_specs=pl.BlockSpec((NBK, F), lambda i: (i, 0)),
        out_shape=jax.ShapeDtypeStruct((N_PAD, F), jnp.float32),
    )(xcat, w_embed)


def _node_update_body(agg_ref, h_ref, w_big_ref, g_ref, gl0_ref, rep_ref, w_read_ref,
                      nf_ref, l0_ref, preds_ref):
    agg = agg_ref[...] * (1.0 / AVG_NEI)
    sc = jnp.dot(h_ref[...], w_big_ref[...], preferred_element_type=jnp.float32)
    nf = agg + sc
    inv = jnp.dot(nf * nf, g_ref[...], preferred_element_type=jnp.float32)
    en = jnp.exp(-2.0 * inv)
    gate = 1.0 + 0.1 * ((1.0 - en) / (1.0 + en))
    gbig = jnp.dot(gate, rep_ref[...], preferred_element_type=jnp.float32)
    nfn = nf * gbig
    nf_ref[...] = nfn
    l0_ref[...] = jnp.dot(nfn, gl0_ref[...], preferred_element_type=jnp.float32)
    preds_ref[...] = jnp.dot(nfn, w_read_ref[...], preferred_element_type=jnp.float32)


def _node_update(agg_raw, h_in, w_big, g_mat, gl0_mat, rep_mat, w_read):
    kin = h_in.shape[1]
    return pl.pallas_call(
        _node_update_body,
        grid=(N_PAD // NBK,),
        in_specs=[
            pl.BlockSpec((NBK, F * L2), lambda i: (i, 0)),
            pl.BlockSpec((NBK, kin), lambda i: (i, 0)),
            pl.BlockSpec((kin, F * L2), lambda i: (0, 0)),
            pl.BlockSpec((F * L2, F), lambda i: (0, 0)),
            pl.BlockSpec((F * L2, F), lambda i: (0, 0)),
            pl.BlockSpec((F, F * L2), lambda i: (0, 0)),
            pl.BlockSpec((F * L2, NOUT), lambda i: (0, 0)),
        ],
        out_specs=[
            pl.BlockSpec((NBK, F * L2), lambda i: (i, 0)),
            pl.BlockSpec((NBK, F), lambda i: (i, 0)),
            pl.BlockSpec((NBK, NOUT), lambda i: (i, 0)),
        ],
        out_shape=[
            jax.ShapeDtypeStruct((N_PAD, F * L2), jnp.float32),
            jax.ShapeDtypeStruct((N_PAD, F), jnp.float32),
            jax.ShapeDtypeStruct((N_PAD, NOUT), jnp.float32),
        ],
    )(agg_raw, h_in, w_big, g_mat, gl0_mat, rep_mat, w_read)


def _loss_body(preds_ref, eps_ref, out_ref):
    i = pl.program_id(0)

    @pl.when(i == 0)
    def _():
        out_ref[...] = jnp.zeros_like(out_ref)

    err = (preds_ref[...] - eps_ref[...]) ** 2
    out_ref[...] += jnp.sum(err, axis=0, keepdims=True)


def _loss_sums(preds_sum, eps_pad):
    return pl.pallas_call(
        _loss_body,
        grid=(N_PAD // NBK,),
        in_specs=[
            pl.BlockSpec((NBK, NOUT), lambda i: (i, 0)),
            pl.BlockSpec((NBK, NOUT), lambda i: (i, 0)),
        ],
        out_specs=pl.BlockSpec((1, NOUT), lambda i: (0, 0)),
        out_shape=jax.ShapeDtypeStruct((1, NOUT), jnp.float32),
    )(preds_sum, eps_pad)


def kernel(positions, node_attrs, shifts, eps, w_embed, w_r1_0, w_r2_0, w_r1_1, w_r2_1,
           w_sc_0, w_sc_1, w_read_0, w_read_1, edge_index, batch):
    alphas = 1.0 - jnp.linspace(1e-4, 0.02, T)
    abar = jnp.cumprod(alphas)[T_IDX]
    s = jnp.sqrt(abar)
    sq = jnp.sqrt(1.0 - abar)
    node_attrs = node_attrs / 4.0
    pos_n = s * positions + sq * eps[:, -3:]
    attrs_n = s * node_attrs + sq * eps[:, :NE]
    t_feat = jnp.full((N, 1), T_IDX / float(T), dtype=jnp.float32)
    xcat = jnp.concatenate([attrs_n, t_feat], axis=-1)
    xcat_pad = jnp.zeros((N_PAD, NE + 1), jnp.float32).at[:N].set(xcat)
    h0_tbl = _h0_embed(xcat_pad, w_embed)

    sender = edge_index[0]
    receiver = edge_index[1]
    pad = E_PAD - E
    snd_pad = jnp.concatenate([sender, jnp.zeros((pad,), jnp.int32)])
    rcv_pad = jnp.concatenate([receiver, jnp.zeros((pad,), jnp.int32)])
    rcv_scatter = jnp.concatenate([receiver, jnp.full((pad,), 2 * HALF, jnp.int32)])

    pos_tbl = jnp.zeros((N, 16), jnp.float32).at[:, :3].set(pos_n)
    pos_s16, pos_r16 = _gather_geom(pos_tbl, snd_pad, rcv_pad)

    eye_f = jnp.eye(F, dtype=jnp.float32)
    eye_l = jnp.eye(L2, dtype=jnp.float32)
    rep_mat = jnp.kron(eye_f, jnp.ones((1, L2), jnp.float32))
    g_mat = rep_mat.T
    e0 = jnp.zeros((1, L2), jnp.float32).at[0, 0].set(1.0)
    gl0_mat = jnp.kron(eye_f, e0).T
    w_big_1 = jnp.kron(w_sc_1, eye_l)
    w0_big = jnp.kron(w_sc_0, e0)
    zeros_acc = jnp.zeros((ACC_ROWS, F * L2), jnp.float32)

    h_gather = _gather_h(h0_tbl, snd_pad)
    m = _edge_messages(pos_s16, pos_r16, h_gather, w_r1_0, w_r2_0)
    agg_raw = _scatter_add(m, rcv_scatter, zeros_acc)
    nf1, l0_1, preds0 = _node_update(agg_raw, h0_tbl, w0_big, g_mat, gl0_mat, rep_mat, w_read_0)

    h_gather = _gather_h(l0_1, snd_pad)
    m = _edge_messages(pos_s16, pos_r16, h_gather, w_r1_1, w_r2_1)
    agg_raw = _scatter_add(m, rcv_scatter, zeros_acc)
    _, _, preds1 = _node_update(agg_raw, nf1, w_big_1, g_mat, gl0_mat, rep_mat, w_read_1)

    preds_pad = preds0 + preds1
    eps_pad = jnp.zeros((N_PAD, NOUT), jnp.float32).at[:N].set(eps)
    err_sums = _loss_sums(preds_pad, eps_pad)[0]
    preds_sum = preds_pad[:N]

    pn_pos = preds_sum[:, -3:]
    pn_lab = preds_sum[:, :-3]
    loss = (0.5 * jnp.sum(err_sums) / (float(N) * (3.0 + NE))).reshape(1)
    return (pn_lab, pn_pos, eps[:, :NE], eps[:, -3:], loss)
